# baseline (speedup 1.0000x reference)
import functools

import jax
import jax.numpy as jnp
from jax import lax
from jax.experimental import pallas as pl
from jax.experimental.pallas import tpu as pltpu
from jax.experimental.pallas import tpu_sc as plsc

EPS = 0.1
NC, NS, L = 2, 16, 16
NW = NC * NS
CH = 128


def _sc_mesh():
    return plsc.VectorSubcoreMesh(core_axis_name="c", subcore_axis_name="s",
                                  num_cores=NC, num_subcores=NS)


def _make_deg(NT, NCH):
    SPT = NT // NS
    SB = 8
    NSUP = NCH // SB

    @functools.partial(
        pl.kernel,
        out_type=jax.ShapeDtypeStruct((NC * NT,), jnp.float32),
        mesh=_sc_mesh(),
        scratch_types=[pltpu.VMEM((SPT,), jnp.float32),
                       pltpu.VMEM((2, SB, CH), jnp.int32),
                       pltpu.VMEM((CH,), jnp.float32),
                       pltpu.VMEM_SHARED((NT,), jnp.float32),
                       pltpu.SemaphoreType.DMA,
                       pltpu.SemaphoreType.DMA,
                       pltpu.SemaphoreType.DMA,
                       pltpu.SemaphoreType.DMA],
    )
    def deg_kernel(col_hbm, out_hbm, zeros_v, idxc, ones_v, sdeg,
                   sf0, sf1, sa0, sa1):
        cid = lax.axis_index("c")
        sid = lax.axis_index("s")
        wid = cid * NS + sid
        sf = (sf0, sf1)
        sa = (sa0, sa1)

        zero = jnp.zeros((L,), jnp.float32)
        one = jnp.ones((L,), jnp.float32)

        def zbody(i, c):
            zeros_v[pl.ds(i * L, L)] = zero
            return c
        lax.fori_loop(0, SPT // L, zbody, 0)
        for g in range(CH // L):
            ones_v[pl.ds(g * L, L)] = one

        pltpu.sync_copy(zeros_v, sdeg.at[pl.ds(sid * SPT, SPT)])
        plsc.subcore_barrier()

        sbase = wid * NSUP
        pltpu.async_copy(col_hbm.at[pl.ds(sbase * SB, SB)], idxc.at[0], sf[0])

        def stage(s, b):
            pltpu.make_async_copy(col_hbm.at[pl.ds(sbase * SB, SB)],
                                  idxc.at[b], sf[b]).wait()
            for r in range(SB):
                pltpu.async_copy(ones_v, sdeg.at[idxc.at[b, r]], sa[b],
                                 add=True)

            @pl.when(s + 1 < NSUP)
            def _():
                @pl.when(s >= 1)
                def _():
                    for r in range(SB):
                        pltpu.make_async_copy(
                            ones_v, sdeg.at[idxc.at[1 - b, r]],
                            sa[1 - b]).wait()
                pltpu.async_copy(col_hbm.at[pl.ds((sbase + s + 1) * SB, SB)],
                                 idxc.at[1 - b], sf[1 - b])

        def pair(s2, c):
            stage(2 * s2, 0)
            stage(2 * s2 + 1, 1)
            return c
        lax.fori_loop(0, NSUP // 2, pair, 0)
        for r in range(SB):
            pltpu.make_async_copy(ones_v, sdeg.at[idxc.at[1, r]],
                                  sa[1]).wait()
        plsc.subcore_barrier()

        pltpu.sync_copy(sdeg.at[pl.ds(sid * SPT, SPT)],
                        out_hbm.at[pl.ds(cid * NT + sid * SPT, SPT)])

    return deg_kernel


def _make_stats(N, D, BLK):
    def body(x_ref, degt_ref, attl_ref, attr_ref,
             al_ref, ar_ref, dis_ref, cs_ref):
        xb = x_ref[...]
        al = jnp.sum(xb * attl_ref[...], axis=1, keepdims=True)
        ar = jnp.sum(xb * attr_ref[...], axis=1, keepdims=True)
        tot = jnp.sum(degt_ref[...], axis=1, keepdims=True) + 1.0
        al_ref[...] = al
        ar_ref[...] = ar
        dis_ref[...] = lax.rsqrt(tot)
        cs_ref[...] = jnp.tanh(al + ar) / tot + EPS

    return pl.pallas_call(
        body,
        grid=(N // BLK,),
        in_specs=[pl.BlockSpec((BLK, D), lambda i: (i, 0)),
                  pl.BlockSpec((BLK, NC), lambda i: (i, 0)),
                  pl.BlockSpec((1, D), lambda i: (0, 0)),
                  pl.BlockSpec((1, D), lambda i: (0, 0))],
        out_specs=[pl.BlockSpec((BLK, 1), lambda i: (i, 0))] * 4,
        out_shape=[jax.ShapeDtypeStruct((N, 1), jnp.float32)] * 4,
    )


def _make_prep(NT, NCH):
    PB = 4
    NST = NCH // PB

    @functools.partial(
        pl.kernel,
        out_type=jax.ShapeDtypeStruct((NW * NCH, CH), jnp.float32),
        mesh=_sc_mesh(),
        compiler_params=pltpu.CompilerParams(needs_layout_passes=False),
        scratch_types=[
            pltpu.VMEM((NT,), jnp.float32),
            pltpu.VMEM((NT,), jnp.float32),
            pltpu.VMEM((NT,), jnp.float32),
            pltpu.VMEM((2, PB, CH), jnp.int32),
            pltpu.VMEM((2, PB, CH), jnp.int32),
            pltpu.VMEM((2, PB, CH), jnp.float32),
            pltpu.SemaphoreType.DMA,
            pltpu.SemaphoreType.DMA,
            pltpu.SemaphoreType.DMA,
            pltpu.SemaphoreType.DMA,
        ],
    )
    def prep_kernel(rp_hbm, cp_hbm, al_hbm, ar_hbm, dis_hbm, cv_hbm,
                    al_v, ar_v, dis_v, idxr, idxc, cvb, si0, si1, so0, so1):
        cid = lax.axis_index("c")
        sid = lax.axis_index("s")
        wid = cid * NS + sid
        si = (si0, si1)
        so = (so0, so1)

        pltpu.sync_copy(al_hbm, al_v)
        pltpu.sync_copy(ar_hbm, ar_v)
        pltpu.sync_copy(dis_hbm, dis_v)

        cbase = wid * NCH
        for b in range(2):
            pltpu.async_copy(rp_hbm.at[pl.ds(cbase + b * PB, PB)],
                             idxr.at[b], si[b])
            pltpu.async_copy(cp_hbm.at[pl.ds(cbase + b * PB, PB)],
                             idxc.at[b], si[b])

        def stage(s, b):
            pltpu.make_async_copy(rp_hbm.at[pl.ds(cbase, PB)],
                                  idxr.at[b], si[b]).wait()
            pltpu.make_async_copy(cp_hbm.at[pl.ds(cbase, PB)],
                                  idxc.at[b], si[b]).wait()

            @pl.when(s >= 2)
            def _():
                pltpu.make_async_copy(cvb.at[b],
                                      cv_hbm.at[pl.ds(cbase, PB)],
                                      so[b]).wait()

            for r in range(PB):
                def gbody(g, c):
                    sl = pl.ds(g * L, L)
                    rg = idxr[b, r, sl]
                    cg = idxc[b, r, sl]
                    av = plsc.load_gather(al_v, [rg])
                    bv = plsc.load_gather(ar_v, [cg])
                    dr = plsc.load_gather(dis_v, [rg])
                    dc = plsc.load_gather(dis_v, [cg])
                    t = 1.0 - 2.0 / (jnp.exp(2.0 * (av + bv)) + 1.0)
                    cvb[b, r, sl] = t * dr * dc
                    return c
                lax.fori_loop(0, CH // L, gbody, 0)

            pltpu.async_copy(cvb.at[b], cv_hbm.at[pl.ds(cbase + s * PB, PB)],
                             so[b])

            @pl.when(s + 2 < NST)
            def _():
                pltpu.async_copy(rp_hbm.at[pl.ds(cbase + (s + 2) * PB, PB)],
                                 idxr.at[b], si[b])
                pltpu.async_copy(cp_hbm.at[pl.ds(cbase + (s + 2) * PB, PB)],
                                 idxc.at[b], si[b])

        def pair(s2, c):
            stage(2 * s2, 0)
            stage(2 * s2 + 1, 1)
            return c
        lax.fori_loop(0, NST // 2, pair, 0)
        pltpu.make_async_copy(cvb.at[0], cv_hbm.at[pl.ds(cbase, PB)],
                              so[0]).wait()
        pltpu.make_async_copy(cvb.at[1], cv_hbm.at[pl.ds(cbase, PB)],
                              so[1]).wait()

    return prep_kernel


def _make_edge(N, D, NT, NCH):
    RPT = NT // NS
    ZR = 80

    @functools.partial(
        pl.kernel,
        out_type=jax.ShapeDtypeStruct((NC * N, D), jnp.float32),
        mesh=_sc_mesh(),
        compiler_params=pltpu.CompilerParams(needs_layout_passes=False),
        scratch_types=[
            pltpu.VMEM((2, 2, CH), jnp.int32),
            pltpu.VMEM((2, CH), jnp.int32),
            pltpu.VMEM((2, CH), jnp.float32),
            pltpu.VMEM((2, CH, D), jnp.float32),
            pltpu.VMEM((ZR, D), jnp.float32),
            pltpu.VMEM_SHARED((NT, D), jnp.float32),
            pltpu.SemaphoreType.DMA,
            pltpu.SemaphoreType.DMA,
            pltpu.SemaphoreType.DMA,
            pltpu.SemaphoreType.DMA,
            pltpu.SemaphoreType.DMA,
            pltpu.SemaphoreType.DMA,
            pltpu.SemaphoreType.DMA,
            pltpu.SemaphoreType.DMA,
        ],
    )
    def edge_kernel(rp_hbm, cp_hbm, cv_hbm, x_hbm, out_hbm, idx, colb, cvb,
                    rows, zbuf, agg, si0, si1, sg0a, sg0b, sg1a, sg1b,
                    ss0, ss1):
        cid = lax.axis_index("c")
        sid = lax.axis_index("s")
        wid = cid * NS + sid
        si = (si0, si1)
        sg = ((sg0a, sg0b), (sg1a, sg1b))
        ss = (ss0, ss1)
        HC = CH // 2

        def gather_start(b):
            pltpu.async_copy(x_hbm.at[idx.at[b, 0, pl.ds(0, HC)]],
                             rows.at[b, pl.ds(0, HC)], sg[b][0])
            pltpu.async_copy(x_hbm.at[idx.at[b, 0, pl.ds(HC, HC)]],
                             rows.at[b, pl.ds(HC, HC)], sg[b][1])

        def gather_wait(b):
            pltpu.make_async_copy(x_hbm.at[idx.at[b, 0, pl.ds(0, HC)]],
                                  rows.at[b, pl.ds(0, HC)], sg[b][0]).wait()
            pltpu.make_async_copy(x_hbm.at[idx.at[b, 0, pl.ds(HC, HC)]],
                                  rows.at[b, pl.ds(HC, HC)], sg[b][1]).wait()

        zero = jnp.zeros((L,), jnp.float32)

        def zb(r, c):
            for j in range(D // L):
                zbuf[r, pl.ds(j * L, L)] = zero
            return c
        lax.fori_loop(0, ZR, zb, 0)

        def za(i, c):
            pltpu.sync_copy(zbuf, agg.at[pl.ds(sid * RPT + i * ZR, ZR)])
            return c
        lax.fori_loop(0, RPT // ZR, za, 0)
        plsc.subcore_barrier()

        cbase = wid * NCH

        for b in range(2):
            pltpu.async_copy(rp_hbm.at[cbase + b], idx.at[b, 0], si[b])
            pltpu.async_copy(cp_hbm.at[cbase + b], idx.at[b, 1], si[b])
            pltpu.async_copy(cv_hbm.at[cbase + b], cvb.at[b], si[b])
        pltpu.make_async_copy(rp_hbm.at[cbase], idx.at[0, 0], si[0]).wait()
        pltpu.make_async_copy(cp_hbm.at[cbase], idx.at[0, 1], si[0]).wait()
        pltpu.make_async_copy(cv_hbm.at[cbase], cvb.at[0], si[0]).wait()
        gather_start(0)

        def stage(k, b):
            gather_wait(b)

            @pl.when(k + 1 < NCH)
            def _():
                pltpu.make_async_copy(rp_hbm.at[cbase], idx.at[1 - b, 0],
                                      si[1 - b]).wait()
                pltpu.make_async_copy(cp_hbm.at[cbase], idx.at[1 - b, 1],
                                      si[1 - b]).wait()
                pltpu.make_async_copy(cv_hbm.at[cbase], cvb.at[1 - b],
                                      si[1 - b]).wait()

                @pl.when(k >= 1)
                def _():
                    pltpu.make_async_copy(rows.at[1 - b],
                                          agg.at[colb.at[1 - b]],
                                          ss[1 - b]).wait()
                gather_start(1 - b)

            def gbody(g, c):
                sl = pl.ds(g * L, L)
                colb[b, sl] = idx[b, 1, sl]
                cv = cvb[b, sl]
                e0 = g * L
                for lane in range(L):
                    s = cv[lane]
                    for j in range(D // L):
                        slj = pl.ds(j * L, L)
                        rows[b, e0 + lane, slj] = rows[b, e0 + lane, slj] * s
                return c
            lax.fori_loop(0, CH // L, gbody, 0)

            @pl.when(k + 2 < NCH)
            def _():
                pltpu.async_copy(rp_hbm.at[cbase + k + 2], idx.at[b, 0], si[b])
                pltpu.async_copy(cp_hbm.at[cbase + k + 2], idx.at[b, 1], si[b])
                pltpu.async_copy(cv_hbm.at[cbase + k + 2], cvb.at[b], si[b])

            pltpu.async_copy(rows.at[b], agg.at[colb.at[b]], ss[b], add=True)

        def pair(k2, c):
            stage(2 * k2, 0)
            stage(2 * k2 + 1, 1)
            return c
        lax.fori_loop(0, NCH // 2, pair, 0)

        pltpu.make_async_copy(rows.at[0], agg.at[colb.at[0]], ss[0]).wait()
        pltpu.make_async_copy(rows.at[1], agg.at[colb.at[1]], ss[1]).wait()
        plsc.subcore_barrier()

        LAST = N - (NS - 1) * RPT

        @pl.when(sid < NS - 1)
        def _():
            pltpu.sync_copy(agg.at[pl.ds(sid * RPT, RPT)],
                            out_hbm.at[pl.ds(cid * N + sid * RPT, RPT)])

        @pl.when(sid == NS - 1)
        def _():
            pltpu.sync_copy(agg.at[pl.ds((NS - 1) * RPT, LAST)],
                            out_hbm.at[pl.ds(cid * N + (NS - 1) * RPT, LAST)])

    return edge_kernel


def _make_mlp(N, D, BLK):
    def body(a0_ref, a1_ref, x_ref, cs_ref, w1_ref, b1_ref, w2_ref, b2_ref,
             o_ref):
        outb = a0_ref[...] + a1_ref[...] + cs_ref[...] * x_ref[...]
        h = lax.dot_general(outb, w1_ref[...], (((1,), (1,)), ((), ())),
                            preferred_element_type=jnp.float32) + b1_ref[...]
        h = jnp.where(h > 0, h, jnp.exp(jnp.minimum(h, 0.0)) - 1.0)
        o_ref[...] = lax.dot_general(h, w2_ref[...], (((1,), (1,)), ((), ())),
                                     preferred_element_type=jnp.float32) \
            + b2_ref[...]

    full = lambda i: (0, 0)
    blk = lambda i: (i, 0)
    nb = N // BLK
    return pl.pallas_call(
        body,
        grid=(nb,),
        in_specs=[pl.BlockSpec((BLK, D), blk),
                  pl.BlockSpec((BLK, D), lambda i: (i + nb, 0)),
                  pl.BlockSpec((BLK, D), blk),
                  pl.BlockSpec((BLK, 1), blk),
                  pl.BlockSpec((D, D), full),
                  pl.BlockSpec((1, D), full),
                  pl.BlockSpec((D, D), full),
                  pl.BlockSpec((1, D), full)],
        out_specs=pl.BlockSpec((BLK, D), blk),
        out_shape=jax.ShapeDtypeStruct((N, D), jnp.float32),
    )


def kernel(x, edge_index, att_l, att_r, W1, b1, W2, b2):
    N, D = x.shape
    E = edge_index.shape[1]
    EPC = NW * CH
    NCH = -(-E // EPC)
    NCH = ((NCH + 15) // 16) * 16
    EP = NCH * EPC
    NT = ((N + 1 + 255) // 256) * 256
    BLK = 2000

    pad = EP - E
    pi = jnp.arange(pad, dtype=edge_index.dtype)
    rowp = jnp.concatenate([edge_index[0], pi % N]).reshape(NW * NCH, CH)
    colp = jnp.concatenate([edge_index[1],
                            N + pi % (NT - N)]).reshape(NW * NCH, CH)

    degp = _make_deg(NT, NCH)(colp)
    degt = jnp.stack([degp[:N], degp[NT:NT + N]], axis=1)
    al, ar, dis, cs = _make_stats(N, D, BLK)(x, degt, att_l, att_r)

    zpad = jnp.zeros((NT - N,), jnp.float32)
    al_t = jnp.concatenate([al[:, 0], zpad])
    ar_t = jnp.concatenate([ar[:, 0], zpad])
    dis_t = jnp.concatenate([dis[:, 0], zpad])

    cv = _make_prep(NT, NCH)(rowp, colp, al_t, ar_t, dis_t)
    aggp = _make_edge(N, D, NT, NCH)(rowp, colp, cv, x)
    return _make_mlp(N, D, BLK)(aggp, aggp, x, cs,
                                W1, b1.reshape(1, D), W2, b2.reshape(1, D))


# speedup vs baseline: 1.0242x; 1.0242x over previous
import functools

import jax
import jax.numpy as jnp
from jax import lax
from jax.experimental import pallas as pl
from jax.experimental.pallas import tpu as pltpu
from jax.experimental.pallas import tpu_sc as plsc

EPS = 0.1
NC, NS, L = 2, 16, 16
NW = NC * NS
CH = 128


def _sc_mesh():
    return plsc.VectorSubcoreMesh(core_axis_name="c", subcore_axis_name="s",
                                  num_cores=NC, num_subcores=NS)


def _make_deg(NT, NCH):
    SPT = NT // NS
    SB = 8
    NSUP = NCH // SB

    @functools.partial(
        pl.kernel,
        out_type=jax.ShapeDtypeStruct((NC * NT,), jnp.float32),
        mesh=_sc_mesh(),
        scratch_types=[pltpu.VMEM((SPT,), jnp.float32),
                       pltpu.VMEM((2, SB, CH), jnp.int32),
                       pltpu.VMEM((CH,), jnp.float32),
                       pltpu.VMEM_SHARED((NT,), jnp.float32),
                       pltpu.SemaphoreType.DMA,
                       pltpu.SemaphoreType.DMA,
                       pltpu.SemaphoreType.DMA,
                       pltpu.SemaphoreType.DMA],
    )
    def deg_kernel(col_hbm, out_hbm, zeros_v, idxc, ones_v, sdeg,
                   sf0, sf1, sa0, sa1):
        cid = lax.axis_index("c")
        sid = lax.axis_index("s")
        wid = cid * NS + sid
        sf = (sf0, sf1)
        sa = (sa0, sa1)

        zero = jnp.zeros((L,), jnp.float32)
        one = jnp.ones((L,), jnp.float32)

        def zbody(i, c):
            zeros_v[pl.ds(i * L, L)] = zero
            return c
        lax.fori_loop(0, SPT // L, zbody, 0)
        for g in range(CH // L):
            ones_v[pl.ds(g * L, L)] = one

        pltpu.sync_copy(zeros_v, sdeg.at[pl.ds(sid * SPT, SPT)])
        plsc.subcore_barrier()

        sbase = wid * NSUP
        pltpu.async_copy(col_hbm.at[pl.ds(sbase * SB, SB)], idxc.at[0], sf[0])

        def stage(s, b):
            pltpu.make_async_copy(col_hbm.at[pl.ds(sbase * SB, SB)],
                                  idxc.at[b], sf[b]).wait()
            for r in range(SB):
                pltpu.async_copy(ones_v, sdeg.at[idxc.at[b, r]], sa[b],
                                 add=True)

            @pl.when(s + 1 < NSUP)
            def _():
                @pl.when(s >= 1)
                def _():
                    for r in range(SB):
                        pltpu.make_async_copy(
                            ones_v, sdeg.at[idxc.at[1 - b, r]],
                            sa[1 - b]).wait()
                pltpu.async_copy(col_hbm.at[pl.ds((sbase + s + 1) * SB, SB)],
                                 idxc.at[1 - b], sf[1 - b])

        def pair(s2, c):
            stage(2 * s2, 0)
            stage(2 * s2 + 1, 1)
            return c
        lax.fori_loop(0, NSUP // 2, pair, 0)
        for r in range(SB):
            pltpu.make_async_copy(ones_v, sdeg.at[idxc.at[1, r]],
                                  sa[1]).wait()
        plsc.subcore_barrier()

        pltpu.sync_copy(sdeg.at[pl.ds(sid * SPT, SPT)],
                        out_hbm.at[pl.ds(cid * NT + sid * SPT, SPT)])

    return deg_kernel


def _make_stats(N, D, BLK):
    def body(x_ref, degt_ref, attl_ref, attr_ref,
             al_ref, ar_ref, dis_ref, cs_ref):
        xb = x_ref[...]
        al = jnp.sum(xb * attl_ref[...], axis=1, keepdims=True)
        ar = jnp.sum(xb * attr_ref[...], axis=1, keepdims=True)
        tot = jnp.sum(degt_ref[...], axis=1, keepdims=True) + 1.0
        al_ref[...] = al
        ar_ref[...] = ar
        dis_ref[...] = lax.rsqrt(tot)
        cs_ref[...] = jnp.tanh(al + ar) / tot + EPS

    return pl.pallas_call(
        body,
        grid=(N // BLK,),
        in_specs=[pl.BlockSpec((BLK, D), lambda i: (i, 0)),
                  pl.BlockSpec((BLK, NC), lambda i: (i, 0)),
                  pl.BlockSpec((1, D), lambda i: (0, 0)),
                  pl.BlockSpec((1, D), lambda i: (0, 0))],
        out_specs=[pl.BlockSpec((BLK, 1), lambda i: (i, 0))] * 4,
        out_shape=[jax.ShapeDtypeStruct((N, 1), jnp.float32)] * 4,
    )


def _make_prep(NT, NCH):
    PB = 4
    NST = NCH // PB

    @functools.partial(
        pl.kernel,
        out_type=jax.ShapeDtypeStruct((NW * NCH, CH), jnp.float32),
        mesh=_sc_mesh(),
        compiler_params=pltpu.CompilerParams(needs_layout_passes=False),
        scratch_types=[
            pltpu.VMEM((NT,), jnp.float32),
            pltpu.VMEM((NT,), jnp.float32),
            pltpu.VMEM((NT,), jnp.float32),
            pltpu.VMEM((2, PB, CH), jnp.int32),
            pltpu.VMEM((2, PB, CH), jnp.int32),
            pltpu.VMEM((2, PB, CH), jnp.float32),
            pltpu.SemaphoreType.DMA,
            pltpu.SemaphoreType.DMA,
            pltpu.SemaphoreType.DMA,
            pltpu.SemaphoreType.DMA,
        ],
    )
    def prep_kernel(rp_hbm, cp_hbm, al_hbm, ar_hbm, dis_hbm, cv_hbm,
                    al_v, ar_v, dis_v, idxr, idxc, cvb, si0, si1, so0, so1):
        cid = lax.axis_index("c")
        sid = lax.axis_index("s")
        wid = cid * NS + sid
        si = (si0, si1)
        so = (so0, so1)

        pltpu.sync_copy(al_hbm, al_v)
        pltpu.sync_copy(ar_hbm, ar_v)
        pltpu.sync_copy(dis_hbm, dis_v)

        cbase = wid * NCH
        for b in range(2):
            pltpu.async_copy(rp_hbm.at[pl.ds(cbase + b * PB, PB)],
                             idxr.at[b], si[b])
            pltpu.async_copy(cp_hbm.at[pl.ds(cbase + b * PB, PB)],
                             idxc.at[b], si[b])

        def stage(s, b):
            pltpu.make_async_copy(rp_hbm.at[pl.ds(cbase, PB)],
                                  idxr.at[b], si[b]).wait()
            pltpu.make_async_copy(cp_hbm.at[pl.ds(cbase, PB)],
                                  idxc.at[b], si[b]).wait()

            @pl.when(s >= 2)
            def _():
                pltpu.make_async_copy(cvb.at[b],
                                      cv_hbm.at[pl.ds(cbase, PB)],
                                      so[b]).wait()

            for r in range(PB):
                def gbody(g, c):
                    sl = pl.ds(g * L, L)
                    rg = idxr[b, r, sl]
                    cg = idxc[b, r, sl]
                    av = plsc.load_gather(al_v, [rg])
                    bv = plsc.load_gather(ar_v, [cg])
                    dr = plsc.load_gather(dis_v, [rg])
                    dc = plsc.load_gather(dis_v, [cg])
                    t = 1.0 - 2.0 / (jnp.exp(2.0 * (av + bv)) + 1.0)
                    cvb[b, r, sl] = t * dr * dc
                    return c
                lax.fori_loop(0, CH // L, gbody, 0)

            pltpu.async_copy(cvb.at[b], cv_hbm.at[pl.ds(cbase + s * PB, PB)],
                             so[b])

            @pl.when(s + 2 < NST)
            def _():
                pltpu.async_copy(rp_hbm.at[pl.ds(cbase + (s + 2) * PB, PB)],
                                 idxr.at[b], si[b])
                pltpu.async_copy(cp_hbm.at[pl.ds(cbase + (s + 2) * PB, PB)],
                                 idxc.at[b], si[b])

        def pair(s2, c):
            stage(2 * s2, 0)
            stage(2 * s2 + 1, 1)
            return c
        lax.fori_loop(0, NST // 2, pair, 0)
        pltpu.make_async_copy(cvb.at[0], cv_hbm.at[pl.ds(cbase, PB)],
                              so[0]).wait()
        pltpu.make_async_copy(cvb.at[1], cv_hbm.at[pl.ds(cbase, PB)],
                              so[1]).wait()

    return prep_kernel


def _make_edge(N, D, NT, NCH):
    RPT = NT // NS
    ZR = 80

    @functools.partial(
        pl.kernel,
        out_type=jax.ShapeDtypeStruct((NC * N, D), jnp.float32),
        mesh=_sc_mesh(),
        compiler_params=pltpu.CompilerParams(needs_layout_passes=False),
        scratch_types=[
            pltpu.VMEM((2, 2, CH), jnp.int32),
            pltpu.VMEM((2, CH), jnp.int32),
            pltpu.VMEM((2, CH), jnp.float32),
            pltpu.VMEM((2, CH, D), jnp.float32),
            pltpu.VMEM((ZR, D), jnp.float32),
            pltpu.VMEM_SHARED((NT, D), jnp.float32),
            pltpu.SemaphoreType.DMA,
            pltpu.SemaphoreType.DMA,
            pltpu.SemaphoreType.DMA,
            pltpu.SemaphoreType.DMA,
            pltpu.SemaphoreType.DMA,
            pltpu.SemaphoreType.DMA,
        ],
    )
    def edge_kernel(rp_hbm, cp_hbm, cv_hbm, x_hbm, out_hbm, idx, colb, cvb,
                    rows, zbuf, agg, si0, si1, sg0, sg1, ss0, ss1):
        cid = lax.axis_index("c")
        sid = lax.axis_index("s")
        wid = cid * NS + sid
        si = (si0, si1)
        sg = (sg0, sg1)
        ss = (ss0, ss1)

        zero = jnp.zeros((L,), jnp.float32)

        def zb(r, c):
            for j in range(D // L):
                zbuf[r, pl.ds(j * L, L)] = zero
            return c
        lax.fori_loop(0, ZR, zb, 0)

        def za(i, c):
            pltpu.sync_copy(zbuf, agg.at[pl.ds(sid * RPT + i * ZR, ZR)])
            return c
        lax.fori_loop(0, RPT // ZR, za, 0)
        plsc.subcore_barrier()

        cbase = wid * NCH

        for b in range(2):
            pltpu.async_copy(rp_hbm.at[cbase + b], idx.at[b, 0], si[b])
            pltpu.async_copy(cp_hbm.at[cbase + b], idx.at[b, 1], si[b])
            pltpu.async_copy(cv_hbm.at[cbase + b], cvb.at[b], si[b])
        pltpu.make_async_copy(rp_hbm.at[cbase], idx.at[0, 0], si[0]).wait()
        pltpu.make_async_copy(cp_hbm.at[cbase], idx.at[0, 1], si[0]).wait()
        pltpu.make_async_copy(cv_hbm.at[cbase], cvb.at[0], si[0]).wait()
        pltpu.async_copy(x_hbm.at[idx.at[0, 0]], rows.at[0], sg[0])

        def stage(k, b):
            pltpu.make_async_copy(x_hbm.at[idx.at[b, 0]], rows.at[b],
                                  sg[b]).wait()

            @pl.when(k + 1 < NCH)
            def _():
                pltpu.make_async_copy(rp_hbm.at[cbase], idx.at[1 - b, 0],
                                      si[1 - b]).wait()
                pltpu.make_async_copy(cp_hbm.at[cbase], idx.at[1 - b, 1],
                                      si[1 - b]).wait()
                pltpu.make_async_copy(cv_hbm.at[cbase], cvb.at[1 - b],
                                      si[1 - b]).wait()

                @pl.when(k >= 1)
                def _():
                    pltpu.make_async_copy(rows.at[1 - b],
                                          agg.at[colb.at[1 - b]],
                                          ss[1 - b]).wait()
                pltpu.async_copy(x_hbm.at[idx.at[1 - b, 0]], rows.at[1 - b],
                                 sg[1 - b])

            def gbody(g, c):
                sl = pl.ds(g * L, L)
                colb[b, sl] = idx[b, 1, sl]
                cv = cvb[b, sl]
                e0 = g * L
                for lane in range(L):
                    s = cv[lane]
                    for j in range(D // L):
                        slj = pl.ds(j * L, L)
                        rows[b, e0 + lane, slj] = rows[b, e0 + lane, slj] * s
                return c
            lax.fori_loop(0, CH // L, gbody, 0)

            @pl.when(k + 2 < NCH)
            def _():
                pltpu.async_copy(rp_hbm.at[cbase + k + 2], idx.at[b, 0], si[b])
                pltpu.async_copy(cp_hbm.at[cbase + k + 2], idx.at[b, 1], si[b])
                pltpu.async_copy(cv_hbm.at[cbase + k + 2], cvb.at[b], si[b])

            pltpu.async_copy(rows.at[b], agg.at[colb.at[b]], ss[b], add=True)

        def pair(k2, c):
            stage(2 * k2, 0)
            stage(2 * k2 + 1, 1)
            return c
        lax.fori_loop(0, NCH // 2, pair, 0)

        pltpu.make_async_copy(rows.at[0], agg.at[colb.at[0]], ss[0]).wait()
        pltpu.make_async_copy(rows.at[1], agg.at[colb.at[1]], ss[1]).wait()
        plsc.subcore_barrier()

        LAST = N - (NS - 1) * RPT

        @pl.when(sid < NS - 1)
        def _():
            pltpu.sync_copy(agg.at[pl.ds(sid * RPT, RPT)],
                            out_hbm.at[pl.ds(cid * N + sid * RPT, RPT)])

        @pl.when(sid == NS - 1)
        def _():
            pltpu.sync_copy(agg.at[pl.ds((NS - 1) * RPT, LAST)],
                            out_hbm.at[pl.ds(cid * N + (NS - 1) * RPT, LAST)])

    return edge_kernel


def _make_mlp(N, D, BLK):
    def body(a0_ref, a1_ref, x_ref, cs_ref, w1_ref, b1_ref, w2_ref, b2_ref,
             o_ref):
        outb = a0_ref[...] + a1_ref[...] + cs_ref[...] * x_ref[...]
        h = lax.dot_general(outb, w1_ref[...], (((1,), (1,)), ((), ())),
                            preferred_element_type=jnp.float32) + b1_ref[...]
        h = jnp.where(h > 0, h, jnp.exp(jnp.minimum(h, 0.0)) - 1.0)
        o_ref[...] = lax.dot_general(h, w2_ref[...], (((1,), (1,)), ((), ())),
                                     preferred_element_type=jnp.float32) \
            + b2_ref[...]

    full = lambda i: (0, 0)
    blk = lambda i: (i, 0)
    nb = N // BLK
    return pl.pallas_call(
        body,
        grid=(nb,),
        in_specs=[pl.BlockSpec((BLK, D), blk),
                  pl.BlockSpec((BLK, D), lambda i: (i + nb, 0)),
                  pl.BlockSpec((BLK, D), blk),
                  pl.BlockSpec((BLK, 1), blk),
                  pl.BlockSpec((D, D), full),
                  pl.BlockSpec((1, D), full),
                  pl.BlockSpec((D, D), full),
                  pl.BlockSpec((1, D), full)],
        out_specs=pl.BlockSpec((BLK, D), blk),
        out_shape=jax.ShapeDtypeStruct((N, D), jnp.float32),
    )


def kernel(x, edge_index, att_l, att_r, W1, b1, W2, b2):
    N, D = x.shape
    E = edge_index.shape[1]
    EPC = NW * CH
    NCH = -(-E // EPC)
    NCH = ((NCH + 15) // 16) * 16
    EP = NCH * EPC
    NT = ((N + 1 + 255) // 256) * 256
    BLK = 2000

    pad = EP - E
    pi = jnp.arange(pad, dtype=edge_index.dtype)
    rowp = jnp.concatenate([edge_index[0], pi % N]).reshape(NW * NCH, CH)
    colp = jnp.concatenate([edge_index[1],
                            N + pi % (NT - N)]).reshape(NW * NCH, CH)

    degp = _make_deg(NT, NCH)(colp)
    degt = jnp.stack([degp[:N], degp[NT:NT + N]], axis=1)
    al, ar, dis, cs = _make_stats(N, D, BLK)(x, degt, att_l, att_r)

    zpad = jnp.zeros((NT - N,), jnp.float32)
    al_t = jnp.concatenate([al[:, 0], zpad])
    ar_t = jnp.concatenate([ar[:, 0], zpad])
    dis_t = jnp.concatenate([dis[:, 0], zpad])

    cv = _make_prep(NT, NCH)(rowp, colp, al_t, ar_t, dis_t)
    aggp = _make_edge(N, D, NT, NCH)(rowp, colp, cv, x)
    return _make_mlp(N, D, BLK)(aggp, aggp, x, cs,
                                W1, b1.reshape(1, D), W2, b2.reshape(1, D))


# speedup vs baseline: 1.0246x; 1.0004x over previous
import functools

import jax
import jax.numpy as jnp
from jax import lax
from jax.experimental import pallas as pl
from jax.experimental.pallas import tpu as pltpu
from jax.experimental.pallas import tpu_sc as plsc

EPS = 0.1
NC, NS, L = 2, 16, 16
NW = NC * NS
CH = 128


def _sc_mesh():
    return plsc.VectorSubcoreMesh(core_axis_name="c", subcore_axis_name="s",
                                  num_cores=NC, num_subcores=NS)


def _make_deg(NT, NCH):
    SPT = NT // NS
    SB = 8
    NSUP = NCH // SB

    @functools.partial(
        pl.kernel,
        out_type=jax.ShapeDtypeStruct((NC * NT,), jnp.float32),
        mesh=_sc_mesh(),
        scratch_types=[pltpu.VMEM((SPT,), jnp.float32),
                       pltpu.VMEM((2, SB, CH), jnp.int32),
                       pltpu.VMEM((CH,), jnp.float32),
                       pltpu.VMEM_SHARED((NT,), jnp.float32),
                       pltpu.SemaphoreType.DMA,
                       pltpu.SemaphoreType.DMA,
                       pltpu.SemaphoreType.DMA,
                       pltpu.SemaphoreType.DMA],
    )
    def deg_kernel(col_hbm, out_hbm, zeros_v, idxc, ones_v, sdeg,
                   sf0, sf1, sa0, sa1):
        cid = lax.axis_index("c")
        sid = lax.axis_index("s")
        wid = cid * NS + sid
        sf = (sf0, sf1)
        sa = (sa0, sa1)

        zero = jnp.zeros((L,), jnp.float32)
        one = jnp.ones((L,), jnp.float32)

        def zbody(i, c):
            zeros_v[pl.ds(i * L, L)] = zero
            return c
        lax.fori_loop(0, SPT // L, zbody, 0)
        for g in range(CH // L):
            ones_v[pl.ds(g * L, L)] = one

        pltpu.sync_copy(zeros_v, sdeg.at[pl.ds(sid * SPT, SPT)])
        plsc.subcore_barrier()

        sbase = wid * NSUP
        pltpu.async_copy(col_hbm.at[pl.ds(sbase * SB, SB)], idxc.at[0], sf[0])

        def stage(s, b):
            pltpu.make_async_copy(col_hbm.at[pl.ds(sbase * SB, SB)],
                                  idxc.at[b], sf[b]).wait()
            for r in range(SB):
                pltpu.async_copy(ones_v, sdeg.at[idxc.at[b, r]], sa[b],
                                 add=True)

            @pl.when(s + 1 < NSUP)
            def _():
                @pl.when(s >= 1)
                def _():
                    for r in range(SB):
                        pltpu.make_async_copy(
                            ones_v, sdeg.at[idxc.at[1 - b, r]],
                            sa[1 - b]).wait()
                pltpu.async_copy(col_hbm.at[pl.ds((sbase + s + 1) * SB, SB)],
                                 idxc.at[1 - b], sf[1 - b])

        def pair(s2, c):
            stage(2 * s2, 0)
            stage(2 * s2 + 1, 1)
            return c
        lax.fori_loop(0, NSUP // 2, pair, 0)
        for r in range(SB):
            pltpu.make_async_copy(ones_v, sdeg.at[idxc.at[1, r]],
                                  sa[1]).wait()
        plsc.subcore_barrier()

        pltpu.sync_copy(sdeg.at[pl.ds(sid * SPT, SPT)],
                        out_hbm.at[pl.ds(cid * NT + sid * SPT, SPT)])

    return deg_kernel


def _make_stats(N, D, BLK):
    def body(x_ref, degt_ref, attl_ref, attr_ref,
             al_ref, ar_ref, dis_ref, cs_ref):
        xb = x_ref[...]
        al = jnp.sum(xb * attl_ref[...], axis=1, keepdims=True)
        ar = jnp.sum(xb * attr_ref[...], axis=1, keepdims=True)
        tot = jnp.sum(degt_ref[...], axis=1, keepdims=True) + 1.0
        al_ref[...] = al
        ar_ref[...] = ar
        dis_ref[...] = lax.rsqrt(tot)
        cs_ref[...] = jnp.tanh(al + ar) / tot + EPS

    return pl.pallas_call(
        body,
        grid=(N // BLK,),
        in_specs=[pl.BlockSpec((BLK, D), lambda i: (i, 0)),
                  pl.BlockSpec((BLK, NC), lambda i: (i, 0)),
                  pl.BlockSpec((1, D), lambda i: (0, 0)),
                  pl.BlockSpec((1, D), lambda i: (0, 0))],
        out_specs=[pl.BlockSpec((BLK, 1), lambda i: (i, 0))] * 4,
        out_shape=[jax.ShapeDtypeStruct((N, 1), jnp.float32)] * 4,
    )


def _make_prep(NT, NCH):
    PB = 4
    NST = NCH // PB

    @functools.partial(
        pl.kernel,
        out_type=jax.ShapeDtypeStruct((NW * NCH, CH), jnp.float32),
        mesh=_sc_mesh(),
        compiler_params=pltpu.CompilerParams(needs_layout_passes=False),
        scratch_types=[
            pltpu.VMEM((NT,), jnp.float32),
            pltpu.VMEM((NT,), jnp.float32),
            pltpu.VMEM((NT,), jnp.float32),
            pltpu.VMEM((2, PB, CH), jnp.int32),
            pltpu.VMEM((2, PB, CH), jnp.int32),
            pltpu.VMEM((2, PB, CH), jnp.float32),
            pltpu.SemaphoreType.DMA,
            pltpu.SemaphoreType.DMA,
            pltpu.SemaphoreType.DMA,
            pltpu.SemaphoreType.DMA,
        ],
    )
    def prep_kernel(rp_hbm, cp_hbm, al_hbm, ar_hbm, dis_hbm, cv_hbm,
                    al_v, ar_v, dis_v, idxr, idxc, cvb, si0, si1, so0, so1):
        cid = lax.axis_index("c")
        sid = lax.axis_index("s")
        wid = cid * NS + sid
        si = (si0, si1)
        so = (so0, so1)

        pltpu.sync_copy(al_hbm, al_v)
        pltpu.sync_copy(ar_hbm, ar_v)
        pltpu.sync_copy(dis_hbm, dis_v)

        cbase = wid * NCH
        for b in range(2):
            pltpu.async_copy(rp_hbm.at[pl.ds(cbase + b * PB, PB)],
                             idxr.at[b], si[b])
            pltpu.async_copy(cp_hbm.at[pl.ds(cbase + b * PB, PB)],
                             idxc.at[b], si[b])

        def stage(s, b):
            pltpu.make_async_copy(rp_hbm.at[pl.ds(cbase, PB)],
                                  idxr.at[b], si[b]).wait()
            pltpu.make_async_copy(cp_hbm.at[pl.ds(cbase, PB)],
                                  idxc.at[b], si[b]).wait()

            @pl.when(s >= 2)
            def _():
                pltpu.make_async_copy(cvb.at[b],
                                      cv_hbm.at[pl.ds(cbase, PB)],
                                      so[b]).wait()

            for r in range(PB):
                def gbody(g, c):
                    sl = pl.ds(g * L, L)
                    rg = idxr[b, r, sl]
                    cg = idxc[b, r, sl]
                    av = plsc.load_gather(al_v, [rg])
                    bv = plsc.load_gather(ar_v, [cg])
                    dr = plsc.load_gather(dis_v, [rg])
                    dc = plsc.load_gather(dis_v, [cg])
                    t = 1.0 - 2.0 / (jnp.exp(2.0 * (av + bv)) + 1.0)
                    cvb[b, r, sl] = t * dr * dc
                    return c
                lax.fori_loop(0, CH // L, gbody, 0)

            pltpu.async_copy(cvb.at[b], cv_hbm.at[pl.ds(cbase + s * PB, PB)],
                             so[b])

            @pl.when(s + 2 < NST)
            def _():
                pltpu.async_copy(rp_hbm.at[pl.ds(cbase + (s + 2) * PB, PB)],
                                 idxr.at[b], si[b])
                pltpu.async_copy(cp_hbm.at[pl.ds(cbase + (s + 2) * PB, PB)],
                                 idxc.at[b], si[b])

        def pair(s2, c):
            stage(2 * s2, 0)
            stage(2 * s2 + 1, 1)
            return c
        lax.fori_loop(0, NST // 2, pair, 0)
        pltpu.make_async_copy(cvb.at[0], cv_hbm.at[pl.ds(cbase, PB)],
                              so[0]).wait()
        pltpu.make_async_copy(cvb.at[1], cv_hbm.at[pl.ds(cbase, PB)],
                              so[1]).wait()

    return prep_kernel


def _make_edge(N, D, NT, NCH):
    RPT = NT // NS
    ZR = 80

    @functools.partial(
        pl.kernel,
        out_type=jax.ShapeDtypeStruct((NC * N, D), jnp.float32),
        mesh=_sc_mesh(),
        compiler_params=pltpu.CompilerParams(needs_layout_passes=False),
        scratch_types=[
            pltpu.VMEM((2, 2, CH), jnp.int32),
            pltpu.VMEM((2, CH), jnp.int32),
            pltpu.VMEM((2, CH), jnp.float32),
            pltpu.VMEM((2, CH, D), jnp.float32),
            pltpu.VMEM((ZR, D), jnp.float32),
            pltpu.VMEM_SHARED((NT, D), jnp.float32),
            pltpu.SemaphoreType.DMA,
            pltpu.SemaphoreType.DMA,
            pltpu.SemaphoreType.DMA,
            pltpu.SemaphoreType.DMA,
            pltpu.SemaphoreType.DMA,
            pltpu.SemaphoreType.DMA,
        ],
    )
    def edge_kernel(rp_hbm, cp_hbm, cv_hbm, x_hbm, out_hbm, idx, colb, cvb,
                    rows, zbuf, agg, si0, si1, sg0, sg1, ss0, ss1):
        cid = lax.axis_index("c")
        sid = lax.axis_index("s")
        wid = cid * NS + sid
        si = (si0, si1)
        sg = (sg0, sg1)
        ss = (ss0, ss1)

        cbase = wid * NCH

        for b in range(2):
            pltpu.async_copy(rp_hbm.at[cbase + b], idx.at[b, 0], si[b])
            pltpu.async_copy(cp_hbm.at[cbase + b], idx.at[b, 1], si[b])
            pltpu.async_copy(cv_hbm.at[cbase + b], cvb.at[b], si[b])

        zero = jnp.zeros((L,), jnp.float32)

        def zb(r, c):
            for j in range(D // L):
                zbuf[r, pl.ds(j * L, L)] = zero
            return c
        lax.fori_loop(0, ZR, zb, 0)

        def za(i, c):
            pltpu.sync_copy(zbuf, agg.at[pl.ds(sid * RPT + i * ZR, ZR)])
            return c
        lax.fori_loop(0, RPT // ZR, za, 0)

        pltpu.make_async_copy(rp_hbm.at[cbase], idx.at[0, 0], si[0]).wait()
        pltpu.make_async_copy(cp_hbm.at[cbase], idx.at[0, 1], si[0]).wait()
        pltpu.make_async_copy(cv_hbm.at[cbase], cvb.at[0], si[0]).wait()
        pltpu.async_copy(x_hbm.at[idx.at[0, 0]], rows.at[0], sg[0])
        plsc.subcore_barrier()

        def stage(k, b):
            pltpu.make_async_copy(x_hbm.at[idx.at[b, 0]], rows.at[b],
                                  sg[b]).wait()

            @pl.when(k + 1 < NCH)
            def _():
                pltpu.make_async_copy(rp_hbm.at[cbase], idx.at[1 - b, 0],
                                      si[1 - b]).wait()
                pltpu.make_async_copy(cp_hbm.at[cbase], idx.at[1 - b, 1],
                                      si[1 - b]).wait()
                pltpu.make_async_copy(cv_hbm.at[cbase], cvb.at[1 - b],
                                      si[1 - b]).wait()

                @pl.when(k >= 1)
                def _():
                    pltpu.make_async_copy(rows.at[1 - b],
                                          agg.at[colb.at[1 - b]],
                                          ss[1 - b]).wait()
                pltpu.async_copy(x_hbm.at[idx.at[1 - b, 0]], rows.at[1 - b],
                                 sg[1 - b])

            def gbody(g, c):
                sl = pl.ds(g * L, L)
                colb[b, sl] = idx[b, 1, sl]
                cv = cvb[b, sl]
                e0 = g * L
                for lane in range(L):
                    s = cv[lane]
                    for j in range(D // L):
                        slj = pl.ds(j * L, L)
                        rows[b, e0 + lane, slj] = rows[b, e0 + lane, slj] * s
                return c
            lax.fori_loop(0, CH // L, gbody, 0)

            @pl.when(k + 2 < NCH)
            def _():
                pltpu.async_copy(rp_hbm.at[cbase + k + 2], idx.at[b, 0], si[b])
                pltpu.async_copy(cp_hbm.at[cbase + k + 2], idx.at[b, 1], si[b])
                pltpu.async_copy(cv_hbm.at[cbase + k + 2], cvb.at[b], si[b])

            pltpu.async_copy(rows.at[b], agg.at[colb.at[b]], ss[b], add=True)

        def pair(k2, c):
            stage(2 * k2, 0)
            stage(2 * k2 + 1, 1)
            return c
        lax.fori_loop(0, NCH // 2, pair, 0)

        pltpu.make_async_copy(rows.at[0], agg.at[colb.at[0]], ss[0]).wait()
        pltpu.make_async_copy(rows.at[1], agg.at[colb.at[1]], ss[1]).wait()
        plsc.subcore_barrier()

        LAST = N - (NS - 1) * RPT

        @pl.when(sid < NS - 1)
        def _():
            pltpu.sync_copy(agg.at[pl.ds(sid * RPT, RPT)],
                            out_hbm.at[pl.ds(cid * N + sid * RPT, RPT)])

        @pl.when(sid == NS - 1)
        def _():
            pltpu.sync_copy(agg.at[pl.ds((NS - 1) * RPT, LAST)],
                            out_hbm.at[pl.ds(cid * N + (NS - 1) * RPT, LAST)])

    return edge_kernel


def _make_mlp(N, D, BLK):
    def body(a0_ref, a1_ref, x_ref, cs_ref, w1_ref, b1_ref, w2_ref, b2_ref,
             o_ref):
        outb = a0_ref[...] + a1_ref[...] + cs_ref[...] * x_ref[...]
        h = lax.dot_general(outb, w1_ref[...], (((1,), (1,)), ((), ())),
                            preferred_element_type=jnp.float32) + b1_ref[...]
        h = jnp.where(h > 0, h, jnp.exp(jnp.minimum(h, 0.0)) - 1.0)
        o_ref[...] = lax.dot_general(h, w2_ref[...], (((1,), (1,)), ((), ())),
                                     preferred_element_type=jnp.float32) \
            + b2_ref[...]

    full = lambda i: (0, 0)
    blk = lambda i: (i, 0)
    nb = N // BLK
    return pl.pallas_call(
        body,
        grid=(nb,),
        in_specs=[pl.BlockSpec((BLK, D), blk),
                  pl.BlockSpec((BLK, D), lambda i: (i + nb, 0)),
                  pl.BlockSpec((BLK, D), blk),
                  pl.BlockSpec((BLK, 1), blk),
                  pl.BlockSpec((D, D), full),
                  pl.BlockSpec((1, D), full),
                  pl.BlockSpec((D, D), full),
                  pl.BlockSpec((1, D), full)],
        out_specs=pl.BlockSpec((BLK, D), blk),
        out_shape=jax.ShapeDtypeStruct((N, D), jnp.float32),
    )


def kernel(x, edge_index, att_l, att_r, W1, b1, W2, b2):
    N, D = x.shape
    E = edge_index.shape[1]
    EPC = NW * CH
    NCH = -(-E // EPC)
    NCH = ((NCH + 15) // 16) * 16
    EP = NCH * EPC
    NT = ((N + 1 + 255) // 256) * 256
    BLK = 2000

    pad = EP - E
    pi = jnp.arange(pad, dtype=edge_index.dtype)
    rowp = jnp.concatenate([edge_index[0], pi % N]).reshape(NW * NCH, CH)
    colp = jnp.concatenate([edge_index[1],
                            N + pi % (NT - N)]).reshape(NW * NCH, CH)

    degp = _make_deg(NT, NCH)(colp)
    degt = jnp.stack([degp[:N], degp[NT:NT + N]], axis=1)
    al, ar, dis, cs = _make_stats(N, D, BLK)(x, degt, att_l, att_r)

    zpad = jnp.zeros((NT - N,), jnp.float32)
    al_t = jnp.concatenate([al[:, 0], zpad])
    ar_t = jnp.concatenate([ar[:, 0], zpad])
    dis_t = jnp.concatenate([dis[:, 0], zpad])

    cv = _make_prep(NT, NCH)(rowp, colp, al_t, ar_t, dis_t)
    aggp = _make_edge(N, D, NT, NCH)(rowp, colp, cv, x)
    return _make_mlp(N, D, BLK)(aggp, aggp, x, cs,
                                W1, b1.reshape(1, D), W2, b2.reshape(1, D))
